# baseline (speedup 1.0000x reference)
import functools
import math

import jax
import jax.numpy as jnp
import numpy as np
from jax.experimental import pallas as pl

B, N = 8, 64
NS, NRS2, NRS3 = 4, 24, 20
RCUT, ETA2, ETA3 = 6.0, 0.32, 2.7
TBD, THREED = 1.8, 0.57
TBW = float(np.sqrt(ETA3 / np.pi) * 13.4)
NPAIRS = NS * (NS + 1) // 2
DAMP = float(np.exp(-(np.pi ** 2) / 2.0))
RS2 = np.linspace(0.0, RCUT, NRS2 + 1)[1:]
RS3 = np.linspace(0.0, RCUT, NRS3 + 1)[1:]
IBLK = 8


def _tables_kernel(x_ref, xt_ref, amm_ref, et_ref,
                   rep2_ref, r_ref, ir_ref, fcm_ref, lp_ref):
    x = x_ref[0]
    xt = xt_ref[0]
    amm = amm_ref[0]
    d2 = jnp.zeros((N, N), jnp.float32)
    for c in range(3):
        col = x[:, c:c + 1]
        row = xt[c:c + 1, :]
        diff = col - row
        d2 = d2 + diff * diff
    rows = jax.lax.broadcasted_iota(jnp.int32, (N, N), 0)
    cols = jax.lax.broadcasted_iota(jnp.int32, (N, N), 1)
    eyef = (rows == cols).astype(jnp.float32)
    r = jnp.sqrt(d2 * (1.0 - eyef) + eyef + 1e-12)
    cmaskf = (1.0 - eyef) * amm * (r < RCUT).astype(jnp.float32)
    fc = 0.5 * (jnp.cos((math.pi / RCUT) * r) + 1.0)
    fcm = fc * cmaskf
    ir = 1.0 / r
    lr = jnp.log(r)
    r_ref[0] = r
    ir_ref[0] = ir
    fcm_ref[0] = fcm
    lp_ref[0] = -THREED * lr

    linv = 1.0 + ETA2 * ir * ir
    loginv = jnp.log(linv)
    mu = lr - 0.5 * loginv
    sig = jnp.sqrt(loginv)
    isig = 1.0 / sig
    pref = fcm * jnp.exp(-TBD * lr) * (1.0 / math.sqrt(2.0 * math.pi)) * isig
    h = -0.5 * isig * isig
    for s in range(NS):
        es = et_ref[0, s:s + 1, :]
        cols_m = []
        for m in range(NRS2):
            lnr2 = float(np.log(RS2[m]))
            rad2m = pref * (1.0 / float(RS2[m])) * jnp.exp(h * (lnr2 - mu) ** 2)
            v = jnp.sum(rad2m * es, axis=1)
            cols_m.append(v.reshape(N, 1))
        rep2_ref[0, :, s * NRS2:(s + 1) * NRS2] = jnp.concatenate(cols_m, axis=1)


def _body3_kernel(r_ref, ir_ref, fcm_ref, lp_ref, et_ref, out_ref):
    ib = pl.program_id(1)
    i0 = ib * IBLK
    rb = r_ref[0, pl.ds(i0, IBLK), :]
    irb = ir_ref[0, pl.ds(i0, IBLK), :]
    fcb = fcm_ref[0, pl.ds(i0, IBLK), :]
    lpb = lp_ref[0, pl.ds(i0, IBLK), :]
    r_full = r_ref[0]
    ir_full = ir_ref[0]
    lp_full = lp_ref[0]

    rij = rb[:, :, None]
    rik = rb[:, None, :]
    rjk = r_full[None, :, :]
    irij = irb[:, :, None]
    irik = irb[:, None, :]
    irjk = ir_full[None, :, :]
    rij2 = rij * rij
    rik2 = rik * rik
    rjk2 = rjk * rjk

    cos_i = (rij2 + rik2 - rjk2) * (0.5 * irij * irik)
    cos_j = (rij2 + rjk2 - rik2) * (0.5 * irij * irjk)
    cos_k = (rik2 + rjk2 - rij2) * (0.5 * irik * irjk)
    ccc = cos_i * cos_j * cos_k
    cl = jnp.clip(cos_i, -1.0 + 1e-6, 1.0 - 1e-6)
    ang0 = (2.0 * DAMP) * cl
    ang1 = (2.0 * DAMP) * jnp.sqrt(1.0 - cl * cl)

    atm_pow = jnp.exp(lpb[:, :, None] + lpb[:, None, :] + lp_full[None, :, :])
    rows = jax.lax.broadcasted_iota(jnp.int32, (N, N), 0)
    cols = jax.lax.broadcasted_iota(jnp.int32, (N, N), 1)
    neq = (rows != cols).astype(jnp.float32)[None, :, :]
    fpair = fcb[:, :, None] * fcb[:, None, :]
    w2 = (0.5 * TBW) * (1.0 + 3.0 * ccc) * atm_pow * neq * fpair
    base0 = w2 * ang0
    base1 = w2 * ang1
    s = 0.5 * (rij + rik)

    for m in range(NRS3):
        dm = s - float(RS3[m])
        r3e = jnp.exp((-ETA3) * dm * dm)
        for c in range(2):
            t = r3e * (base0 if c == 0 else base1)
            for b in range(NS):
                eb = et_ref[0, b:b + 1, :][None, :, :]
                vk = jnp.sum(t * eb, axis=2)
                for a in range(NS):
                    ea = et_ref[0, a:a + 1, :]
                    va = jnp.sum(vk * ea, axis=1)
                    col = ((a * NS + b) * NRS3 + m) * 2 + c
                    out_ref[0, 0, :, col:col + 1] = va.reshape(IBLK, 1)


def kernel(X, Z, atomIDs, molIDs, atom_counts, species, Rs2, Rs3):
    X = X.astype(jnp.float32)
    elem = jnp.argmax(Z[..., None].astype(jnp.float32) == species[None, None, :],
                      axis=-1)
    Et = jax.nn.one_hot(elem, NS, dtype=jnp.float32).transpose(0, 2, 1)
    am = (jnp.arange(N)[None, :] < atom_counts[:, None]).astype(jnp.float32)
    amm = am[:, :, None] * am[:, None, :]
    Xt = X.transpose(0, 2, 1)

    rep2, r, ir, fcm, lp = pl.pallas_call(
        _tables_kernel,
        grid=(B,),
        in_specs=[
            pl.BlockSpec((1, N, 3), lambda b: (b, 0, 0)),
            pl.BlockSpec((1, 3, N), lambda b: (b, 0, 0)),
            pl.BlockSpec((1, N, N), lambda b: (b, 0, 0)),
            pl.BlockSpec((1, NS, N), lambda b: (b, 0, 0)),
        ],
        out_specs=[
            pl.BlockSpec((1, N, NS * NRS2), lambda b: (b, 0, 0)),
            pl.BlockSpec((1, N, N), lambda b: (b, 0, 0)),
            pl.BlockSpec((1, N, N), lambda b: (b, 0, 0)),
            pl.BlockSpec((1, N, N), lambda b: (b, 0, 0)),
            pl.BlockSpec((1, N, N), lambda b: (b, 0, 0)),
        ],
        out_shape=[
            jax.ShapeDtypeStruct((B, N, NS * NRS2), jnp.float32),
            jax.ShapeDtypeStruct((B, N, N), jnp.float32),
            jax.ShapeDtypeStruct((B, N, N), jnp.float32),
            jax.ShapeDtypeStruct((B, N, N), jnp.float32),
            jax.ShapeDtypeStruct((B, N, N), jnp.float32),
        ],
    )(X, Xt, amm, Et)

    nblk = N // IBLK
    ncols = NS * NS * NRS3 * 2
    raw3 = pl.pallas_call(
        _body3_kernel,
        grid=(B, nblk),
        in_specs=[
            pl.BlockSpec((1, N, N), lambda b, ib: (b, 0, 0)),
            pl.BlockSpec((1, N, N), lambda b, ib: (b, 0, 0)),
            pl.BlockSpec((1, N, N), lambda b, ib: (b, 0, 0)),
            pl.BlockSpec((1, N, N), lambda b, ib: (b, 0, 0)),
            pl.BlockSpec((1, NS, N), lambda b, ib: (b, 0, 0)),
        ],
        out_specs=pl.BlockSpec((1, 1, IBLK, ncols), lambda b, ib: (b, ib, 0, 0)),
        out_shape=jax.ShapeDtypeStruct((B, nblk, IBLK, ncols), jnp.float32),
    )(r, ir, fcm, lp, Et)

    raw3 = raw3.reshape(B, N, NS, NS, NRS3, 2)
    parts = []
    for a in range(NS):
        for b in range(a, NS):
            if a == b:
                parts.append(raw3[:, :, a, a])
            else:
                parts.append(raw3[:, :, a, b] + raw3[:, :, b, a])
    rep3 = jnp.stack(parts, axis=2).reshape(B, N, NPAIRS * NRS3 * 2)
    return jnp.concatenate([rep2, rep3], axis=-1)
